# Initial kernel scaffold; baseline (speedup 1.0000x reference)
import jax
import jax.numpy as jnp
from jax.experimental import pallas as pl


def kernel(content_id, part_id, position_table, content_table, part_table):
    raise NotImplementedError("write your pallas kernel here")


# speedup vs baseline: 6.1805x; 6.1805x over previous
import functools

import jax
import jax.numpy as jnp
from jax import lax
from jax.experimental import pallas as pl
from jax.experimental.pallas import tpu as pltpu
from jax.experimental.pallas import tpu_sc as plsc

_BATCH = 4096
_SEQ = 200
_ND = 64
_NPART = 8
_NTOK = _BATCH * _SEQ
_ROW = 128
_NROWS = _NTOK // _ROW
_NW = 32
_ROWS_PER_W = _NROWS // _NW
_RPC = 4
_NCHUNK = _ROWS_PER_W // _RPC
_CTOK = _RPC * _ROW


def _sc_body(ppidx_hbm, cid_hbm, pp_hbm, content_hbm, out_hbm,
             ppi_v, cid_v, buf, sem):
    wid = lax.axis_index("s") * 2 + lax.axis_index("c")

    def chunk_body(c, carry):
        base = wid * _ROWS_PER_W + c * _RPC
        pltpu.sync_copy(ppidx_hbm.at[pl.ds(base, _RPC)], ppi_v)
        pltpu.sync_copy(cid_hbm.at[pl.ds(base, _RPC)], cid_v)
        descs = [
            pltpu.async_copy(pp_hbm.at[ppi_v.at[i]],
                             buf.at[pl.ds(i * _ROW, _ROW)], sem)
            for i in range(_RPC)
        ]
        for d in descs:
            d.wait()
        descs = [
            pltpu.async_copy(content_hbm.at[cid_v.at[i]],
                             buf.at[pl.ds(i * _ROW, _ROW)], sem, add=True)
            for i in range(_RPC)
        ]
        for d in descs:
            d.wait()
        pltpu.sync_copy(buf, out_hbm.at[pl.ds(base * _ROW, _CTOK)])
        return carry

    lax.fori_loop(0, _NCHUNK, chunk_body, 0)


_mesh = plsc.VectorSubcoreMesh(core_axis_name="c", subcore_axis_name="s")

_gather_sum = functools.partial(
    pl.kernel,
    out_type=jax.ShapeDtypeStruct((_NTOK, _ND), jnp.float32),
    mesh=_mesh,
    scratch_types=[
        pltpu.VMEM((_RPC, _ROW), jnp.int32),
        pltpu.VMEM((_RPC, _ROW), jnp.int32),
        pltpu.VMEM((_CTOK, _ND), jnp.float32),
        pltpu.SemaphoreType.DMA,
    ],
    compiler_params=pltpu.CompilerParams(use_tc_tiling_on_sc=False),
)(_sc_body)


def kernel(content_id, part_id, position_table, content_table, part_table):
    pp_table = (position_table[:, None, :]
                + part_table[None, :, :]).reshape(_SEQ * _NPART, _ND)
    pp_idx = (part_id.astype(jnp.int32)
              + (_NPART * jnp.arange(_SEQ, dtype=jnp.int32))[None, :]
              ).reshape(_NROWS, _ROW)
    cid = content_id.astype(jnp.int32).reshape(_NROWS, _ROW)
    out = _gather_sum(pp_idx, cid, pp_table, content_table)
    return out.reshape(_BATCH, _SEQ, _ND)
